# baseline (speedup 1.0000x reference)
import functools

import jax
import jax.numpy as jnp
import numpy as np
from jax import lax
from jax.experimental import pallas as pl
from jax.experimental.pallas import tpu as pltpu
from jax.experimental.pallas import tpu_sc as plsc

B = 32
N = 1_000_000
K = 100_000

_K1 = np.uint32(928981903)
_K2 = np.uint32(3453687069)


NT = 8192
CT = 128
NCHUNK = NT // CT
VPC = CT * 8
NBLK = 4096


def _find_from_top(hist_v, thresh):
    big = jnp.int32(1 << 30)

    def cond(state):
        blk, done, carry, bucket, above = state
        return jnp.logical_and(jnp.logical_not(done), blk >= 0)

    def body(state):
        blk, done, carry, bucket, above = state
        v = hist_v[pl.ds(blk * 16, 16)]
        vr = lax.rev(v, (0,))
        cum = plsc.cumsum(vr) + carry
        mask = cum >= thresh
        found = jnp.any(mask)
        mmin = jnp.min(jnp.where(mask, cum, big))
        isf = jnp.logical_and(jnp.logical_and(mask, cum == mmin), vr > 0)
        lane = jnp.sum(jnp.where(isf, lax.iota(jnp.int32, 16), 0))
        vfirst = jnp.sum(jnp.where(isf, vr, 0))
        bkt = blk * 16 + (15 - lane)
        abv = mmin - vfirst
        newly = jnp.logical_and(found, jnp.logical_not(done))
        bucket = jnp.where(newly, bkt, bucket)
        above = jnp.where(newly, abv, above)
        done = jnp.logical_or(done, found)
        carry = jnp.max(cum)
        return blk - 1, done, bucket, above, carry

    def body_wrap(state):
        blk, done, carry, bucket, above = state
        nblk, ndone, nbucket, nabove, ncarry = body(state)
        return (nblk, ndone, ncarry, nbucket, nabove)

    init = (jnp.int32(NBLK - 1), jnp.bool_(False), jnp.int32(0),
            jnp.int32(0), jnp.int32(0))
    _, _, _, bucket, above = lax.while_loop(cond, body_wrap, init)
    return bucket, above


def _zero_hist(hist_v):
    def zbody(i, _):
        hist_v[pl.ds(i * 16, 16)] = jnp.zeros((16,), jnp.int32)
        return 0
    lax.fori_loop(0, NBLK, zbody, 0)


CW = 65_536


def _linearize_body(l_ref, out_ref):
    j = pl.program_id(1)
    x = l_ref[...]
    bi = lax.bitcast_convert_type(x, jnp.int32)
    m = lax.shift_right_arithmetic(bi, 31)
    s = lax.bitwise_xor(bi, lax.bitwise_and(m, jnp.int32(0x7FFFFFFF)))
    sb = lax.bitwise_xor(s, jnp.int32(-2147483648))
    col = j * CW + lax.broadcasted_iota(jnp.int32, (8, CW), 1)
    sb = jnp.where(col < N, sb, 0)
    out_ref[...] = sb.reshape(8, CW // 128, 128)


def _linearize(logits):
    return pl.pallas_call(
        _linearize_body,
        grid=(B // 8, NT * 128 // CW),
        in_specs=[pl.BlockSpec((8, CW), lambda g, j: (g, j))],
        out_specs=pl.BlockSpec((8, CW // 128, 128), lambda g, j: (g, j, 0)),
        out_shape=jax.ShapeDtypeStruct((B, NT, 128), jnp.int32),
    )(logits)


def _sc_kth_body(logits_hbm, kth_hbm, hist_v, buf0_v, buf1_v, out_v,
                 sem0, sem1):
    cid = lax.axis_index("c")
    sid = lax.axis_index("s")
    row = sid * 2 + cid

    def src(c):
        return logits_hbm.at[row, pl.ds(c * CT, CT), :]

    def stream_pass(vec_body):
        pltpu.async_copy(src(0), buf0_v, sem0)

        def chunk_pair(i, _):
            c0 = 2 * i
            pltpu.async_copy(src(c0 + 1), buf1_v, sem1)
            pltpu.make_async_copy(src(c0), buf0_v, sem0).wait()

            @plsc.parallel_loop(0, VPC, unroll=8)
            def _(j):
                vec_body(buf0_v, j)

            @pl.when(c0 + 2 < NCHUNK)
            def _():
                pltpu.async_copy(src(c0 + 2), buf0_v, sem0)
            pltpu.make_async_copy(src(c0 + 1), buf1_v, sem1).wait()

            @plsc.parallel_loop(0, VPC, unroll=8)
            def _(j):
                vec_body(buf1_v, j)
            return 0
        lax.fori_loop(0, NCHUNK // 2, chunk_pair, 0)

    _zero_hist(hist_v)

    def vec_a(buf, j):
        t = lax.shift_right_logical(j, 3)
        lo16 = lax.bitwise_and(j, 7) * 16
        sb = buf[t, pl.ds(lo16, 16)]
        hi = lax.shift_right_logical(sb, 16)
        cnt, last = plsc.scan_count(hi)
        plsc.addupdate_scatter(hist_v, [hi], cnt, mask=last)
    stream_pass(vec_a)

    bucket_hi, above_hi = _find_from_top(hist_v, jnp.int32(K))

    _zero_hist(hist_v)

    def vec_b(buf, j):
        t = lax.shift_right_logical(j, 3)
        lo16 = lax.bitwise_and(j, 7) * 16
        sb = buf[t, pl.ds(lo16, 16)]
        hi = lax.shift_right_logical(sb, 16)
        lo = lax.bitwise_and(sb, jnp.int32(0xFFFF))
        elig = hi == bucket_hi
        cnt, last = plsc.scan_count(lo, mask=elig)
        plsc.addupdate_scatter(hist_v, [lo], cnt, mask=last)
    stream_pass(vec_b)

    bucket_lo, _ = _find_from_top(hist_v, jnp.int32(K) - above_hi)

    s_kth = lax.bitwise_or(
        lax.shift_left(bucket_hi - jnp.int32(32768), jnp.int32(16)), bucket_lo)
    sv = jnp.full((16,), s_kth, jnp.int32)
    m = lax.shift_right_arithmetic(sv, 31)
    bits = lax.bitwise_xor(sv, lax.bitwise_and(m, jnp.int32(0x7FFFFFFF)))
    kf = lax.bitcast_convert_type(bits, jnp.float32)
    for l in range(8):
        out_v[0, pl.ds(l * 16, 16)] = kf
    pltpu.sync_copy(out_v, kth_hbm.at[row])


@functools.lru_cache(maxsize=1)
def _sc_kth():
    return functools.partial(
        pl.kernel,
        out_type=jax.ShapeDtypeStruct((B, 8, 128), jnp.float32),
        mesh=plsc.VectorSubcoreMesh(core_axis_name="c", subcore_axis_name="s"),
        compiler_params=pltpu.CompilerParams(needs_layout_passes=False),
        scratch_types=[
            pltpu.VMEM((65536,), jnp.int32),
            pltpu.VMEM((CT, 128), jnp.int32),
            pltpu.VMEM((CT, 128), jnp.int32),
            pltpu.VMEM((8, 128), jnp.float32),
            pltpu.SemaphoreType.DMA,
            pltpu.SemaphoreType.DMA,
        ],
    )(_sc_kth_body)


BC = 4096
GC = (N + BC - 1) // BC


def _threefry_bits(n_u32):
    ks0 = _K1
    ks1 = _K2
    ks2 = np.uint32(_K1 ^ _K2 ^ np.uint32(0x1BD11BDA))
    ks = (ks0, ks1, ks2)
    rots = ((13, 15, 26, 6), (17, 29, 16, 24))
    x0 = jnp.full_like(n_u32, ks0)
    x1 = n_u32 + ks1
    for i in range(5):
        for r in rots[i % 2]:
            x0 = x0 + x1
            x1 = lax.bitwise_xor(
                lax.bitwise_or(
                    lax.shift_left(x1, np.uint32(r)),
                    lax.shift_right_logical(x1, np.uint32(32 - r))),
                x0)
        x0 = x0 + ks[(i + 1) % 3]
        x1 = x1 + ks[(i + 2) % 3] + np.uint32(i + 1)
    return lax.bitwise_xor(x0, x1)


def _tc_gumbel_body(l_ref, rn_ref, v_ref):
    step = pl.program_id(0)
    l = l_ref[...]
    col = step * BA + lax.broadcasted_iota(jnp.int32, (B, BA), 1)
    rn = rn_ref[...].astype(jnp.uint32)
    n = rn + col.astype(jnp.uint32)

    bits = _threefry_bits(n)
    fb = lax.bitwise_or(lax.shift_right_logical(bits, np.uint32(9)),
                        np.uint32(0x3F800000))
    u = lax.bitcast_convert_type(fb, jnp.float32) - np.float32(1.0)
    u = jnp.maximum(u, np.float32(np.finfo(np.float32).tiny))
    g = -jnp.log(-jnp.log(u))
    v_ref[...] = l + g


def _tc_gumbel(logits):
    rown = (jnp.arange(B, dtype=jnp.int32) * N).reshape(B, 1)
    return pl.pallas_call(
        _tc_gumbel_body,
        grid=(GA,),
        in_specs=[
            pl.BlockSpec((B, BA), lambda i: (0, i)),
            pl.BlockSpec((B, 1), lambda i: (0, 0)),
        ],
        out_specs=pl.BlockSpec((B, BA), lambda i: (0, i)),
        out_shape=jax.ShapeDtypeStruct((B, GA * BA), jnp.float32),
    )(logits, rown)


BA = 16384
GA = (N + BA - 1) // BA


def _tc_argmax_body(l_ref, v_ref, kth_ref, out_ref, mx_s, ix_s):
    step = pl.program_id(0)

    @pl.when(step == 0)
    def _():
        mx_s[...] = jnp.full((B, 1), -jnp.inf, jnp.float32)
        ix_s[...] = jnp.full((B, 1), 1 << 30, jnp.int32)

    l = l_ref[...]
    v = v_ref[...]
    kth = kth_ref[...]
    col = step * BA + lax.broadcasted_iota(jnp.int32, (B, BA), 1)

    valid = jnp.logical_and(l >= kth, col < N)
    v = jnp.where(valid, v, -jnp.inf)

    m = jnp.max(v, axis=1, keepdims=True)
    idx = jnp.min(jnp.where(v == m, col, 1 << 30), axis=1, keepdims=True)

    cur = mx_s[...]
    curi = ix_s[...]
    better = m > cur
    tie = m == cur
    mx_s[...] = jnp.maximum(cur, m)
    ix_s[...] = jnp.where(better, idx,
                          jnp.where(tie, jnp.minimum(curi, idx), curi))

    @pl.when(step == GA - 1)
    def _():
        out_ref[...] = ix_s[...]


def _tc_argmax(logits, v, kth):
    return pl.pallas_call(
        _tc_argmax_body,
        grid=(GA,),
        in_specs=[
            pl.BlockSpec((B, BA), lambda i: (0, i)),
            pl.BlockSpec((B, BA), lambda i: (0, i)),
            pl.BlockSpec((B, 1), lambda i: (0, 0)),
        ],
        out_specs=pl.BlockSpec((B, 1), lambda i: (0, 0)),
        out_shape=jax.ShapeDtypeStruct((B, 1), jnp.int32),
        scratch_shapes=[
            pltpu.VMEM((B, 1), jnp.float32),
            pltpu.VMEM((B, 1), jnp.int32),
        ],
    )(logits, v, kth)


def kernel(logits):
    lin = _linearize(logits)
    kth_o = _sc_kth()(lin)
    v = _tc_gumbel(logits)
    kth = kth_o[:, 0, :1]
    return _tc_argmax(logits, v, kth)


# speedup vs baseline: 1.0491x; 1.0491x over previous
import functools

import jax
import jax.numpy as jnp
import numpy as np
from jax import lax
from jax.experimental import pallas as pl
from jax.experimental.pallas import tpu as pltpu
from jax.experimental.pallas import tpu_sc as plsc

B = 32
N = 1_000_000
K = 100_000

_K1 = np.uint32(928981903)
_K2 = np.uint32(3453687069)


NT = 8192
CT = 128
NCHUNK = NT // CT
VPC = CT * 8
NBLK = 4096


def _find_from_top(hist_v, thresh):
    big = jnp.int32(1 << 30)

    def cond(state):
        blk, done, carry, bucket, above = state
        return jnp.logical_and(jnp.logical_not(done), blk >= 0)

    def body(state):
        blk, done, carry, bucket, above = state
        v = hist_v[pl.ds(blk * 16, 16)]
        vr = lax.rev(v, (0,))
        cum = plsc.cumsum(vr) + carry
        mask = cum >= thresh
        found = jnp.any(mask)
        mmin = jnp.min(jnp.where(mask, cum, big))
        isf = jnp.logical_and(jnp.logical_and(mask, cum == mmin), vr > 0)
        lane = jnp.sum(jnp.where(isf, lax.iota(jnp.int32, 16), 0))
        vfirst = jnp.sum(jnp.where(isf, vr, 0))
        bkt = blk * 16 + (15 - lane)
        abv = mmin - vfirst
        newly = jnp.logical_and(found, jnp.logical_not(done))
        bucket = jnp.where(newly, bkt, bucket)
        above = jnp.where(newly, abv, above)
        done = jnp.logical_or(done, found)
        carry = jnp.max(cum)
        return blk - 1, done, bucket, above, carry

    def body_wrap(state):
        blk, done, carry, bucket, above = state
        nblk, ndone, nbucket, nabove, ncarry = body(state)
        return (nblk, ndone, ncarry, nbucket, nabove)

    init = (jnp.int32(NBLK - 1), jnp.bool_(False), jnp.int32(0),
            jnp.int32(0), jnp.int32(0))
    _, _, _, bucket, above = lax.while_loop(cond, body_wrap, init)
    return bucket, above


def _zero_hist(hist_v):
    def zbody(i, _):
        hist_v[pl.ds(i * 16, 16)] = jnp.zeros((16,), jnp.int32)
        return 0
    lax.fori_loop(0, NBLK, zbody, 0)


CW = 65_536


def _linearize_body(l_ref, out_ref):
    j = pl.program_id(1)
    x = l_ref[...]
    bi = lax.bitcast_convert_type(x, jnp.int32)
    m = lax.shift_right_arithmetic(bi, 31)
    s = lax.bitwise_xor(bi, lax.bitwise_and(m, jnp.int32(0x7FFFFFFF)))
    sb = lax.bitwise_xor(s, jnp.int32(-2147483648))
    col = j * CW + lax.broadcasted_iota(jnp.int32, (8, CW), 1)
    sb = jnp.where(col < N, sb, 0)
    out_ref[...] = sb.reshape(8, CW // 128, 128)


def _linearize(logits):
    return pl.pallas_call(
        _linearize_body,
        grid=(B // 8, NT * 128 // CW),
        in_specs=[pl.BlockSpec((8, CW), lambda g, j: (g, j))],
        out_specs=pl.BlockSpec((8, CW // 128, 128), lambda g, j: (g, j, 0)),
        out_shape=jax.ShapeDtypeStruct((B, NT, 128), jnp.int32),
    )(logits)


def _sc_kth_body(logits_hbm, kth_hbm, hist_v, buf0_v, buf1_v, out_v,
                 sem0, sem1):
    cid = lax.axis_index("c")
    sid = lax.axis_index("s")
    row = sid * 2 + cid

    def src(c):
        return logits_hbm.at[row, pl.ds(c * CT, CT), :]

    def stream_pass(vec_body):
        pltpu.async_copy(src(0), buf0_v, sem0)

        def chunk_pair(i, _):
            c0 = 2 * i
            pltpu.async_copy(src(c0 + 1), buf1_v, sem1)
            pltpu.make_async_copy(src(c0), buf0_v, sem0).wait()

            @plsc.parallel_loop(0, VPC, unroll=8)
            def _(j):
                vec_body(buf0_v, j)

            @pl.when(c0 + 2 < NCHUNK)
            def _():
                pltpu.async_copy(src(c0 + 2), buf0_v, sem0)
            pltpu.make_async_copy(src(c0 + 1), buf1_v, sem1).wait()

            @plsc.parallel_loop(0, VPC, unroll=8)
            def _(j):
                vec_body(buf1_v, j)
            return 0
        lax.fori_loop(0, NCHUNK // 2, chunk_pair, 0)

    _zero_hist(hist_v)

    def vec_a(buf, j):
        t = lax.shift_right_logical(j, 3)
        lo16 = lax.bitwise_and(j, 7) * 16
        sb = buf[t, pl.ds(lo16, 16)]
        hi = lax.shift_right_logical(sb, 16)
        cnt, last = plsc.scan_count(hi)
        plsc.addupdate_scatter(hist_v, [hi], cnt, mask=last)
    stream_pass(vec_a)

    bucket_hi, above_hi = _find_from_top(hist_v, jnp.int32(K))

    _zero_hist(hist_v)

    def vec_b(buf, j):
        t = lax.shift_right_logical(j, 3)
        lo16 = lax.bitwise_and(j, 7) * 16
        sb = buf[t, pl.ds(lo16, 16)]
        hi = lax.shift_right_logical(sb, 16)
        lo = lax.bitwise_and(sb, jnp.int32(0xFFFF))
        elig = hi == bucket_hi
        cnt, last = plsc.scan_count(lo, mask=elig)
        plsc.addupdate_scatter(hist_v, [lo], cnt, mask=last)
    stream_pass(vec_b)

    bucket_lo, _ = _find_from_top(hist_v, jnp.int32(K) - above_hi)

    s_kth = lax.bitwise_or(
        lax.shift_left(bucket_hi - jnp.int32(32768), jnp.int32(16)), bucket_lo)
    sv = jnp.full((16,), s_kth, jnp.int32)
    m = lax.shift_right_arithmetic(sv, 31)
    bits = lax.bitwise_xor(sv, lax.bitwise_and(m, jnp.int32(0x7FFFFFFF)))
    kf = lax.bitcast_convert_type(bits, jnp.float32)
    for l in range(8):
        out_v[0, pl.ds(l * 16, 16)] = kf
    pltpu.sync_copy(out_v, kth_hbm.at[row])


@functools.lru_cache(maxsize=1)
def _sc_kth():
    return functools.partial(
        pl.kernel,
        out_type=jax.ShapeDtypeStruct((B, 8, 128), jnp.float32),
        mesh=plsc.VectorSubcoreMesh(core_axis_name="c", subcore_axis_name="s"),
        compiler_params=pltpu.CompilerParams(needs_layout_passes=False),
        scratch_types=[
            pltpu.VMEM((65536,), jnp.int32),
            pltpu.VMEM((CT, 128), jnp.int32),
            pltpu.VMEM((CT, 128), jnp.int32),
            pltpu.VMEM((8, 128), jnp.float32),
            pltpu.SemaphoreType.DMA,
            pltpu.SemaphoreType.DMA,
        ],
    )(_sc_kth_body)


BC = 4096
GC = (N + BC - 1) // BC


def _threefry_bits(n_u32):
    ks0 = _K1
    ks1 = _K2
    ks2 = np.uint32(_K1 ^ _K2 ^ np.uint32(0x1BD11BDA))
    ks = (ks0, ks1, ks2)
    rots = ((13, 15, 26, 6), (17, 29, 16, 24))
    x0 = jnp.full_like(n_u32, ks0)
    x1 = n_u32 + ks1
    for i in range(5):
        for r in rots[i % 2]:
            x0 = x0 + x1
            x1 = lax.bitwise_xor(
                lax.bitwise_or(
                    lax.shift_left(x1, np.uint32(r)),
                    lax.shift_right_logical(x1, np.uint32(32 - r))),
                x0)
        x0 = x0 + ks[(i + 1) % 3]
        x1 = x1 + ks[(i + 2) % 3] + np.uint32(i + 1)
    return lax.bitwise_xor(x0, x1)


def _tc_gumbel_body(l_ref, rn_ref, v_ref):
    step = pl.program_id(0)
    l = l_ref[...]
    col = step * BC + lax.broadcasted_iota(jnp.int32, (B, BC), 1)
    rn = rn_ref[...].astype(jnp.uint32)
    n = rn + col.astype(jnp.uint32)

    bits = _threefry_bits(n)
    fb = lax.bitwise_or(lax.shift_right_logical(bits, np.uint32(9)),
                        np.uint32(0x3F800000))
    u = lax.bitcast_convert_type(fb, jnp.float32) - np.float32(1.0)
    u = jnp.maximum(u, np.float32(np.finfo(np.float32).tiny))
    g = -jnp.log(-jnp.log(u))
    v_ref[...] = l + g


def _tc_gumbel(logits):
    rown = (jnp.arange(B, dtype=jnp.int32) * N).reshape(B, 1)
    return pl.pallas_call(
        _tc_gumbel_body,
        grid=(GC,),
        in_specs=[
            pl.BlockSpec((B, BC), lambda i: (0, i)),
            pl.BlockSpec((B, 1), lambda i: (0, 0)),
        ],
        out_specs=pl.BlockSpec((B, BC), lambda i: (0, i)),
        out_shape=jax.ShapeDtypeStruct((B, GC * BC), jnp.float32),
    )(logits, rown)


BA = 16384
GA = (N + BA - 1) // BA


def _tc_argmax_body(l_ref, v_ref, kth_ref, out_ref, mx_s, ix_s):
    step = pl.program_id(0)

    @pl.when(step == 0)
    def _():
        mx_s[...] = jnp.full((B, 1), -jnp.inf, jnp.float32)
        ix_s[...] = jnp.full((B, 1), 1 << 30, jnp.int32)

    l = l_ref[...]
    v = v_ref[...]
    kth = kth_ref[...]
    col = step * BA + lax.broadcasted_iota(jnp.int32, (B, BA), 1)

    valid = jnp.logical_and(l >= kth, col < N)
    v = jnp.where(valid, v, -jnp.inf)

    m = jnp.max(v, axis=1, keepdims=True)
    idx = jnp.min(jnp.where(v == m, col, 1 << 30), axis=1, keepdims=True)

    cur = mx_s[...]
    curi = ix_s[...]
    better = m > cur
    tie = m == cur
    mx_s[...] = jnp.maximum(cur, m)
    ix_s[...] = jnp.where(better, idx,
                          jnp.where(tie, jnp.minimum(curi, idx), curi))

    @pl.when(step == GA - 1)
    def _():
        out_ref[...] = ix_s[...]


def _tc_argmax(logits, v, kth):
    return pl.pallas_call(
        _tc_argmax_body,
        grid=(GA,),
        in_specs=[
            pl.BlockSpec((B, BA), lambda i: (0, i)),
            pl.BlockSpec((B, BA), lambda i: (0, i)),
            pl.BlockSpec((B, 1), lambda i: (0, 0)),
        ],
        out_specs=pl.BlockSpec((B, 1), lambda i: (0, 0)),
        out_shape=jax.ShapeDtypeStruct((B, 1), jnp.int32),
        scratch_shapes=[
            pltpu.VMEM((B, 1), jnp.float32),
            pltpu.VMEM((B, 1), jnp.int32),
        ],
    )(logits, v, kth)


def kernel(logits):
    lin = _linearize(logits)
    kth_o = _sc_kth()(lin)
    v = _tc_gumbel(logits)
    kth = kth_o[:, 0, :1]
    return _tc_argmax(logits, v, kth)


# speedup vs baseline: 1.5377x; 1.4657x over previous
import functools

import jax
import jax.numpy as jnp
import numpy as np
from jax import lax
from jax.experimental import pallas as pl
from jax.experimental.pallas import tpu as pltpu
from jax.experimental.pallas import tpu_sc as plsc

B = 32
N = 1_000_000
K = 100_000

_K1 = np.uint32(928981903)
_K2 = np.uint32(3453687069)


NT = 8192
CT = 128
NCHUNK = NT // CT
VPC = CT * 8
NBLK = 4096


def _find_from_top(hist_v, thresh):
    big = jnp.int32(1 << 30)

    def cond(state):
        blk, done, carry, bucket, above = state
        return jnp.logical_and(jnp.logical_not(done), blk >= 0)

    def body(state):
        blk, done, carry, bucket, above = state
        v = hist_v[pl.ds(blk * 16, 16)]
        vr = lax.rev(v, (0,))
        cum = plsc.cumsum(vr) + carry
        mask = cum >= thresh
        found = jnp.any(mask)
        mmin = jnp.min(jnp.where(mask, cum, big))
        isf = jnp.logical_and(jnp.logical_and(mask, cum == mmin), vr > 0)
        lane = jnp.sum(jnp.where(isf, lax.iota(jnp.int32, 16), 0))
        vfirst = jnp.sum(jnp.where(isf, vr, 0))
        bkt = blk * 16 + (15 - lane)
        abv = mmin - vfirst
        newly = jnp.logical_and(found, jnp.logical_not(done))
        bucket = jnp.where(newly, bkt, bucket)
        above = jnp.where(newly, abv, above)
        done = jnp.logical_or(done, found)
        carry = jnp.max(cum)
        return blk - 1, done, bucket, above, carry

    def body_wrap(state):
        blk, done, carry, bucket, above = state
        nblk, ndone, nbucket, nabove, ncarry = body(state)
        return (nblk, ndone, ncarry, nbucket, nabove)

    init = (jnp.int32(NBLK - 1), jnp.bool_(False), jnp.int32(0),
            jnp.int32(0), jnp.int32(0))
    _, _, _, bucket, above = lax.while_loop(cond, body_wrap, init)
    return bucket, above


def _zero_hist(hist_v):
    def zbody(i, _):
        hist_v[pl.ds(i * 16, 16)] = jnp.zeros((16,), jnp.int32)
        return 0
    lax.fori_loop(0, NBLK, zbody, 0)


CW = 65_536


def _linearize_body(l_ref, out_ref):
    j = pl.program_id(1)
    x = l_ref[...]
    bi = lax.bitcast_convert_type(x, jnp.int32)
    m = lax.shift_right_arithmetic(bi, 31)
    s = lax.bitwise_xor(bi, lax.bitwise_and(m, jnp.int32(0x7FFFFFFF)))
    sb = lax.bitwise_xor(s, jnp.int32(-2147483648))
    col = j * CW + lax.broadcasted_iota(jnp.int32, (8, CW), 1)
    sb = jnp.where(col < N, sb, 0)
    out_ref[...] = sb.reshape(8, CW // 128, 128)


def _linearize(logits):
    return pl.pallas_call(
        _linearize_body,
        grid=(B // 8, NT * 128 // CW),
        in_specs=[pl.BlockSpec((8, CW), lambda g, j: (g, j))],
        out_specs=pl.BlockSpec((8, CW // 128, 128), lambda g, j: (g, j, 0)),
        out_shape=jax.ShapeDtypeStruct((B, NT, 128), jnp.int32),
    )(logits)


def _sc_kth_body(logits_hbm, kth_hbm, hist_v, buf0_v, buf1_v, out_v,
                 sem0, sem1):
    cid = lax.axis_index("c")
    sid = lax.axis_index("s")
    row = sid * 2 + cid

    def src(c):
        return logits_hbm.at[row, pl.ds(c * CT, CT), :]

    def stream_pass(vec_body):
        pltpu.async_copy(src(0), buf0_v, sem0)

        def chunk_pair(i, _):
            c0 = 2 * i
            pltpu.async_copy(src(c0 + 1), buf1_v, sem1)
            pltpu.make_async_copy(src(c0), buf0_v, sem0).wait()

            @plsc.parallel_loop(0, VPC, unroll=8)
            def _(j):
                vec_body(buf0_v, j)

            @pl.when(c0 + 2 < NCHUNK)
            def _():
                pltpu.async_copy(src(c0 + 2), buf0_v, sem0)
            pltpu.make_async_copy(src(c0 + 1), buf1_v, sem1).wait()

            @plsc.parallel_loop(0, VPC, unroll=8)
            def _(j):
                vec_body(buf1_v, j)
            return 0
        lax.fori_loop(0, NCHUNK // 2, chunk_pair, 0)

    _zero_hist(hist_v)

    def vec_a(buf, j):
        t = lax.shift_right_logical(j, 3)
        lo16 = lax.bitwise_and(j, 7) * 16
        sb = buf[t, pl.ds(lo16, 16)]
        hi = lax.shift_right_logical(sb, 16)
        cnt, last = plsc.scan_count(hi)
        plsc.addupdate_scatter(hist_v, [hi], cnt, mask=last)
    stream_pass(vec_a)

    bucket_hi, above_hi = _find_from_top(hist_v, jnp.int32(K))

    _zero_hist(hist_v)

    def vec_b(buf, j):
        t = lax.shift_right_logical(j, 3)
        lo16 = lax.bitwise_and(j, 7) * 16
        sb = buf[t, pl.ds(lo16, 16)]
        hi = lax.shift_right_logical(sb, 16)
        lo = lax.bitwise_and(sb, jnp.int32(0xFFFF))
        elig = hi == bucket_hi
        cnt, last = plsc.scan_count(lo, mask=elig)
        plsc.addupdate_scatter(hist_v, [lo], cnt, mask=last)
    stream_pass(vec_b)

    bucket_lo, _ = _find_from_top(hist_v, jnp.int32(K) - above_hi)

    s_kth = lax.bitwise_or(
        lax.shift_left(bucket_hi - jnp.int32(32768), jnp.int32(16)), bucket_lo)
    sv = jnp.full((16,), s_kth, jnp.int32)
    m = lax.shift_right_arithmetic(sv, 31)
    bits = lax.bitwise_xor(sv, lax.bitwise_and(m, jnp.int32(0x7FFFFFFF)))
    kf = lax.bitcast_convert_type(bits, jnp.float32)
    for l in range(8):
        out_v[0, pl.ds(l * 16, 16)] = kf
    pltpu.sync_copy(out_v, kth_hbm.at[row])


@functools.lru_cache(maxsize=1)
def _sc_kth():
    return functools.partial(
        pl.kernel,
        out_type=jax.ShapeDtypeStruct((B, 8, 128), jnp.float32),
        mesh=plsc.VectorSubcoreMesh(core_axis_name="c", subcore_axis_name="s"),
        compiler_params=pltpu.CompilerParams(needs_layout_passes=False),
        scratch_types=[
            pltpu.VMEM((65536,), jnp.int32),
            pltpu.VMEM((CT, 128), jnp.int32),
            pltpu.VMEM((CT, 128), jnp.int32),
            pltpu.VMEM((8, 128), jnp.float32),
            pltpu.SemaphoreType.DMA,
            pltpu.SemaphoreType.DMA,
        ],
    )(_sc_kth_body)


BC = 4096
GC = (N + BC - 1) // BC


def _threefry_bits(n_u32):
    ks0 = _K1
    ks1 = _K2
    ks2 = np.uint32(_K1 ^ _K2 ^ np.uint32(0x1BD11BDA))
    ks = (ks0, ks1, ks2)
    rots = ((13, 15, 26, 6), (17, 29, 16, 24))
    x0 = jnp.full_like(n_u32, ks0)
    x1 = n_u32 + ks1
    for i in range(5):
        for r in rots[i % 2]:
            x0 = x0 + x1
            x1 = lax.bitwise_xor(
                lax.bitwise_or(
                    lax.shift_left(x1, np.uint32(r)),
                    lax.shift_right_logical(x1, np.uint32(32 - r))),
                x0)
        x0 = x0 + ks[(i + 1) % 3]
        x1 = x1 + ks[(i + 2) % 3] + np.uint32(i + 1)
    return lax.bitwise_xor(x0, x1)


def _tc_gumbel_body(l_ref, v_ref):
    step = pl.program_id(0)
    l = l_ref[...]
    col = step * BC + lax.broadcasted_iota(jnp.int32, (B, BC), 1)
    rowid = lax.broadcasted_iota(jnp.int32, (B, BC), 0)
    n = (rowid * N + col).astype(jnp.uint32)

    bits = _threefry_bits(n)
    fb = lax.bitwise_or(lax.shift_right_logical(bits, np.uint32(9)),
                        np.uint32(0x3F800000))
    u = lax.bitcast_convert_type(fb, jnp.float32) - np.float32(1.0)
    u = jnp.maximum(u, np.float32(np.finfo(np.float32).tiny))
    g = -jnp.log(-jnp.log(u))
    v_ref[...] = l + g


def _tc_gumbel(logits):
    return pl.pallas_call(
        _tc_gumbel_body,
        grid=(GC,),
        in_specs=[pl.BlockSpec((B, BC), lambda i: (0, i))],
        out_specs=pl.BlockSpec((B, BC), lambda i: (0, i)),
        out_shape=jax.ShapeDtypeStruct((B, GC * BC), jnp.float32),
    )(logits)


BA = 16384
GA = (N + BA - 1) // BA


def _tc_argmax_body(l_ref, v_ref, kth_ref, out_ref, mx_s, ix_s):
    step = pl.program_id(0)

    @pl.when(step == 0)
    def _():
        mx_s[...] = jnp.full((B, 1), -jnp.inf, jnp.float32)
        ix_s[...] = jnp.full((B, 1), 1 << 30, jnp.int32)

    l = l_ref[...]
    v = v_ref[...]
    kth = kth_ref[...]
    col = step * BA + lax.broadcasted_iota(jnp.int32, (B, BA), 1)

    valid = jnp.logical_and(l >= kth, col < N)
    v = jnp.where(valid, v, -jnp.inf)

    m = jnp.max(v, axis=1, keepdims=True)
    idx = jnp.min(jnp.where(v == m, col, 1 << 30), axis=1, keepdims=True)

    cur = mx_s[...]
    curi = ix_s[...]
    better = m > cur
    tie = m == cur
    mx_s[...] = jnp.maximum(cur, m)
    ix_s[...] = jnp.where(better, idx,
                          jnp.where(tie, jnp.minimum(curi, idx), curi))

    @pl.when(step == GA - 1)
    def _():
        out_ref[...] = ix_s[...]


def _tc_argmax(logits, v, kth):
    return pl.pallas_call(
        _tc_argmax_body,
        grid=(GA,),
        in_specs=[
            pl.BlockSpec((B, BA), lambda i: (0, i)),
            pl.BlockSpec((B, BA), lambda i: (0, i)),
            pl.BlockSpec((B, 1), lambda i: (0, 0)),
        ],
        out_specs=pl.BlockSpec((B, 1), lambda i: (0, 0)),
        out_shape=jax.ShapeDtypeStruct((B, 1), jnp.int32),
        scratch_shapes=[
            pltpu.VMEM((B, 1), jnp.float32),
            pltpu.VMEM((B, 1), jnp.int32),
        ],
    )(logits, v, kth)


def kernel(logits):
    lin = _linearize(logits)
    kth_o = _sc_kth()(lin)
    v = _tc_gumbel(logits)
    kth = kth_o[:, 0, :1]
    return _tc_argmax(logits, v, kth)
